# baseline (speedup 1.0000x reference)
import jax
import jax.numpy as jnp
from jax import lax
from jax.experimental import pallas as pl
from jax.experimental.pallas import tpu as pltpu
from jax.experimental.pallas import tpu_sc as plsc

N = 10000
E = 160000
D_IN = 256
D_H = 512
HALF = 128
NP = 10240
WIN = 96
EP = 165888
NROWS = EP // WIN
NC, NS = 2, 16
SUB = NP // NS
WSUB = NROWS // NS
WDEG = NROWS // (NC * NS)
NACC = 10112
SACC = NACC // NS
NPASS = 3
WPASS = WSUB // NPASS

_MESH = plsc.VectorSubcoreMesh(
    core_axis_name="c", subcore_axis_name="s", num_cores=NC, num_subcores=NS)


def _deg_body(edges_hbm, deg_hbm, idxs, ones_v, zeros_v, acc, sem):
  c = lax.axis_index("c")
  s = lax.axis_index("s")
  for k in range(WIN // 16):
    ones_v[pl.ds(k * 16, 16)] = jnp.ones((16,), jnp.float32)
  for k in range(SUB // 16):
    zeros_v[pl.ds(k * 16, 16)] = jnp.zeros((16,), jnp.float32)
  wid = c * NS + s
  pltpu.sync_copy(edges_hbm.at[wid], idxs)
  pltpu.sync_copy(zeros_v, acc.at[pl.ds(s * SUB, SUB)])
  plsc.subcore_barrier()

  cps = [pltpu.async_copy(ones_v, acc.at[idxs.at[w]], sem, add=True)
         for w in range(WDEG)]
  for cp in cps:
    cp.wait()

  plsc.subcore_barrier()
  pltpu.sync_copy(acc.at[pl.ds(s * SUB, SUB)],
                  deg_hbm.at[pl.ds(c * NP + s * SUB, SUB)])


_deg = pl.kernel(
    _deg_body,
    out_type=jax.ShapeDtypeStruct((NC * NP,), jnp.float32),
    mesh=_MESH,
    scratch_types=[
        pltpu.VMEM((WDEG, WIN), jnp.int32),
        pltpu.VMEM((WIN,), jnp.float32),
        pltpu.VMEM((SUB,), jnp.float32),
        pltpu.VMEM_SHARED((NP,), jnp.float32),
        pltpu.SemaphoreType.DMA,
    ],
)


def _agg_body(edges_hbm, xs0_hbm, xs1_hbm, y_hbm, idxs, r0, r1, r2,
              acc, g0, g1, g2, s0, s1, s2):
  c = lax.axis_index("c")
  s = lax.axis_index("s")
  rows = (r0, r1, r2)
  gsem = (g0, g1, g2)
  ssem = (s0, s1, s2)

  for k, xs_hbm in ((0, xs0_hbm), (1, xs1_hbm)):
    @pl.when(c == k)
    def _core():
      pltpu.sync_copy(xs_hbm.at[pl.ds(s * SACC, SACC), :],
                      acc.at[pl.ds(s * SACC, SACC), :])
      plsc.subcore_barrier()

      def gather(w, b):
        return pltpu.async_copy(xs_hbm.at[idxs.at[0, w]], rows[b], gsem[b])

      def scatter(w, b):
        return pltpu.async_copy(rows[b], acc.at[idxs.at[1, w]], ssem[b],
                                add=True)

      def wait_gather(w, b):
        pltpu.make_async_copy(xs_hbm.at[idxs.at[0, w]], rows[b],
                              gsem[b]).wait()

      def wait_scatter(w, b):
        pltpu.make_async_copy(rows[b], acc.at[idxs.at[1, w]], ssem[b]).wait()

      nblk = WPASS // 3

      @pl.loop(0, NPASS)
      def _pass(p):
        @pl.when(p > 0)
        def _():
          wait_scatter(0, 0)
          wait_scatter(0, 1)
          wait_scatter(0, 2)
        pltpu.sync_copy(edges_hbm.at[s * NPASS + p], idxs)
        gather(0, 0)
        gather(1, 1)

        @pl.loop(0, nblk)
        def _block(j):
          w0 = 3 * j
          @pl.when(j > 0)
          def _():
            wait_scatter(w0 - 1, 2)
          gather(w0 + 2, 2)
          wait_gather(w0, 0)
          scatter(w0, 0)
          wait_gather(w0 + 1, 1)
          scatter(w0 + 1, 1)
          @pl.when(j < nblk - 1)
          def _():
            wait_scatter(w0, 0)
            gather(w0 + 3, 0)
            wait_scatter(w0 + 1, 1)
            gather(w0 + 4, 1)
          wait_gather(w0 + 2, 2)
          scatter(w0 + 2, 2)

      wait_scatter(WPASS - 3, 0)
      wait_scatter(WPASS - 2, 1)
      wait_scatter(WPASS - 1, 2)
      plsc.subcore_barrier()

  pltpu.sync_copy(acc.at[pl.ds(s * SACC, SACC), :],
                  y_hbm.at[pl.ds(c * NACC + s * SACC, SACC), :])


_agg = pl.kernel(
    _agg_body,
    out_type=jax.ShapeDtypeStruct((NC * NACC, HALF), jnp.float32),
    mesh=_MESH,
    scratch_types=[
        pltpu.VMEM((2, WPASS, WIN), jnp.int32),
        pltpu.VMEM((WIN, HALF), jnp.float32),
        pltpu.VMEM((WIN, HALF), jnp.float32),
        pltpu.VMEM((WIN, HALF), jnp.float32),
        pltpu.VMEM_SHARED((NACC, HALF), jnp.float32),
        pltpu.SemaphoreType.DMA,
        pltpu.SemaphoreType.DMA,
        pltpu.SemaphoreType.DMA,
        pltpu.SemaphoreType.DMA,
        pltpu.SemaphoreType.DMA,
        pltpu.SemaphoreType.DMA,
    ],
)

_BLK = 256


def _scale_body(x_ref, deg_ref, xs0_ref, xs1_ref, dinv_ref):
  d = deg_ref[0, :] + deg_ref[1, :] + 1.0
  di = lax.rsqrt(d)
  xs = x_ref[...] * di[:, None]
  xs0_ref[...] = xs[:, :HALF]
  xs1_ref[...] = xs[:, HALF:]
  dinv_ref[...] = di


def _scale(x, deg2):
  return pl.pallas_call(
      _scale_body,
      grid=(NP // _BLK,),
      in_specs=[
          pl.BlockSpec((_BLK, D_IN), lambda i: (i, 0)),
          pl.BlockSpec((2, _BLK), lambda i: (0, i)),
      ],
      out_specs=[
          pl.BlockSpec((_BLK, HALF), lambda i: (i, 0)),
          pl.BlockSpec((_BLK, HALF), lambda i: (i, 0)),
          pl.BlockSpec((_BLK,), lambda i: (i,)),
      ],
      out_shape=[
          jax.ShapeDtypeStruct((NP, HALF), jnp.float32),
          jax.ShapeDtypeStruct((NP, HALF), jnp.float32),
          jax.ShapeDtypeStruct((NP,), jnp.float32),
      ],
  )(x, deg2)


def _matmul_body(y_ref, dinv_ref, w_ref, b_ref, a_ref, out_ref):
  di = dinv_ref[...][:, None]
  y0 = y_ref[0] * di
  y1 = y_ref[1] * di
  h = jnp.dot(y0, w_ref[:HALF, :], preferred_element_type=jnp.float32)
  h = h + jnp.dot(y1, w_ref[HALF:, :], preferred_element_type=jnp.float32)
  h = h + b_ref[...][None, :]
  out_ref[...] = jnp.where(h > 0, h, a_ref[...][None, :] * h)


def _matmul(y, dinv, W, b, prelu_a):
  return pl.pallas_call(
      _matmul_body,
      grid=(NP // _BLK,),
      in_specs=[
          pl.BlockSpec((2, _BLK, HALF), lambda i: (0, i, 0)),
          pl.BlockSpec((_BLK,), lambda i: (i,)),
          pl.BlockSpec((D_IN, D_H), lambda i: (0, 0)),
          pl.BlockSpec((D_H,), lambda i: (0,)),
          pl.BlockSpec((D_H,), lambda i: (0,)),
      ],
      out_specs=pl.BlockSpec((_BLK, D_H), lambda i: (i, 0)),
      out_shape=jax.ShapeDtypeStruct((N, D_H), jnp.float32),
  )(y, dinv, W, b, prelu_a)


@jax.jit
def kernel(x, edge_index, W, b, prelu_a):
  src = edge_index[0].astype(jnp.int32)
  dst = edge_index[1].astype(jnp.int32)
  npad = EP - E
  ar = jnp.arange(npad, dtype=jnp.int32)
  psrc = (ar * 911) % N
  pdst = N + (ar % (NACC - N))
  src_pad = jnp.concatenate([src, psrc])
  dst_pad = jnp.concatenate([dst, pdst])
  e5 = jnp.stack([src_pad, dst_pad]).reshape(2, NS, NPASS, WPASS, WIN)
  eagg = jnp.transpose(e5, (1, 2, 0, 3, 4)).reshape(NS * NPASS, 2, WPASS, WIN)
  edeg = dst_pad.reshape(NC * NS, WDEG, WIN)

  deg = _deg(edeg).reshape(2, NP)
  xs0, xs1, dinv = _scale(x, deg)
  y = _agg(eagg, xs0, xs1).reshape(2, NACC, HALF)
  return _matmul(y, dinv, W, b, prelu_a)


# speedup vs baseline: 1.1493x; 1.1493x over previous
import jax
import jax.numpy as jnp
from jax import lax
from jax.experimental import pallas as pl
from jax.experimental.pallas import tpu as pltpu
from jax.experimental.pallas import tpu_sc as plsc

N = 10000
E = 160000
D_IN = 256
D_H = 512
HALF = 128
NP = 10240
WIN = 96
EP = 165888
NROWS = EP // WIN
NC, NS = 2, 16
SUB = NP // NS
WSUB = NROWS // NS
WDEG = NROWS // (NC * NS)
NACC = 10112
SACC = NACC // NS
NPASS = 3
WPASS = WSUB // NPASS

_MESH = plsc.VectorSubcoreMesh(
    core_axis_name="c", subcore_axis_name="s", num_cores=NC, num_subcores=NS)


def _deg_body(edges_hbm, deg_hbm, idxs, ones_v, zeros_v, acc, sem):
  c = lax.axis_index("c")
  s = lax.axis_index("s")
  for k in range(WIN // 16):
    ones_v[pl.ds(k * 16, 16)] = jnp.ones((16,), jnp.float32)
  for k in range(SUB // 16):
    zeros_v[pl.ds(k * 16, 16)] = jnp.zeros((16,), jnp.float32)
  wid = c * NS + s
  pltpu.sync_copy(edges_hbm.at[wid], idxs)
  pltpu.sync_copy(zeros_v, acc.at[pl.ds(s * SUB, SUB)])
  plsc.subcore_barrier()

  cps = [pltpu.async_copy(ones_v, acc.at[idxs.at[w]], sem, add=True)
         for w in range(WDEG)]
  for cp in cps:
    cp.wait()

  plsc.subcore_barrier()
  pltpu.sync_copy(acc.at[pl.ds(s * SUB, SUB)],
                  deg_hbm.at[pl.ds(c * NP + s * SUB, SUB)])


_deg = pl.kernel(
    _deg_body,
    out_type=jax.ShapeDtypeStruct((NC * NP,), jnp.float32),
    mesh=_MESH,
    scratch_types=[
        pltpu.VMEM((WDEG, WIN), jnp.int32),
        pltpu.VMEM((WIN,), jnp.float32),
        pltpu.VMEM((SUB,), jnp.float32),
        pltpu.VMEM_SHARED((NP,), jnp.float32),
        pltpu.SemaphoreType.DMA,
    ],
)


def _agg_body(edges_hbm, xs0_hbm, xs1_hbm, y_hbm, idxs, r0, r1, r2,
              acc, g0, g1, g2, s0, s1, s2):
  c = lax.axis_index("c")
  s = lax.axis_index("s")
  rows = (r0, r1, r2)
  gsem = (g0, g1, g2)
  ssem = (s0, s1, s2)

  for k, xs_hbm in ((0, xs0_hbm), (1, xs1_hbm)):
    @pl.when(c == k)
    def _core():
      pltpu.sync_copy(xs_hbm.at[pl.ds(s * SACC, SACC), :],
                      acc.at[pl.ds(s * SACC, SACC), :])
      plsc.subcore_barrier()

      def gather(w, b):
        return pltpu.async_copy(xs_hbm.at[idxs.at[0, w]], rows[b], gsem[b])

      def scatter(w, b):
        return pltpu.async_copy(rows[b], acc.at[idxs.at[1, w]], ssem[b],
                                add=True)

      def wait_gather(w, b):
        pltpu.make_async_copy(xs_hbm.at[idxs.at[0, w]], rows[b],
                              gsem[b]).wait()

      def wait_scatter(w, b):
        pltpu.make_async_copy(rows[b], acc.at[idxs.at[1, w]], ssem[b]).wait()

      nblk = WPASS // 3

      @pl.loop(0, NPASS)
      def _pass(p):
        @pl.when(p > 0)
        def _():
          wait_scatter(WPASS - 1, 2)
        pltpu.sync_copy(edges_hbm.at[s * NPASS + p], idxs)
        gather(0, 0)
        gather(1, 1)

        @pl.loop(0, nblk)
        def _block(j):
          w0 = 3 * j
          @pl.when(j > 0)
          def _():
            wait_scatter(w0 - 1, 2)
          wait_gather(w0, 0)
          scatter(w0, 0)
          gather(w0 + 2, 2)
          wait_scatter(w0, 0)
          wait_gather(w0 + 1, 1)
          scatter(w0 + 1, 1)
          @pl.when(j < nblk - 1)
          def _():
            gather(w0 + 3, 0)
          wait_scatter(w0 + 1, 1)
          wait_gather(w0 + 2, 2)
          scatter(w0 + 2, 2)
          @pl.when(j < nblk - 1)
          def _():
            gather(w0 + 4, 1)

      wait_scatter(WPASS - 1, 2)
      plsc.subcore_barrier()

  pltpu.sync_copy(acc.at[pl.ds(s * SACC, SACC), :],
                  y_hbm.at[pl.ds(c * NACC + s * SACC, SACC), :])


_agg = pl.kernel(
    _agg_body,
    out_type=jax.ShapeDtypeStruct((NC * NACC, HALF), jnp.float32),
    mesh=_MESH,
    scratch_types=[
        pltpu.VMEM((2, WPASS, WIN), jnp.int32),
        pltpu.VMEM((WIN, HALF), jnp.float32),
        pltpu.VMEM((WIN, HALF), jnp.float32),
        pltpu.VMEM((WIN, HALF), jnp.float32),
        pltpu.VMEM_SHARED((NACC, HALF), jnp.float32),
        pltpu.SemaphoreType.DMA,
        pltpu.SemaphoreType.DMA,
        pltpu.SemaphoreType.DMA,
        pltpu.SemaphoreType.DMA,
        pltpu.SemaphoreType.DMA,
        pltpu.SemaphoreType.DMA,
    ],
)

_BLK = 256


def _scale_body(x_ref, deg_ref, xs0_ref, xs1_ref, dinv_ref):
  d = deg_ref[0, :] + deg_ref[1, :] + 1.0
  di = lax.rsqrt(d)
  xs = x_ref[...] * di[:, None]
  xs0_ref[...] = xs[:, :HALF]
  xs1_ref[...] = xs[:, HALF:]
  dinv_ref[...] = di


def _scale(x, deg2):
  return pl.pallas_call(
      _scale_body,
      grid=(NP // _BLK,),
      in_specs=[
          pl.BlockSpec((_BLK, D_IN), lambda i: (i, 0)),
          pl.BlockSpec((2, _BLK), lambda i: (0, i)),
      ],
      out_specs=[
          pl.BlockSpec((_BLK, HALF), lambda i: (i, 0)),
          pl.BlockSpec((_BLK, HALF), lambda i: (i, 0)),
          pl.BlockSpec((_BLK,), lambda i: (i,)),
      ],
      out_shape=[
          jax.ShapeDtypeStruct((NP, HALF), jnp.float32),
          jax.ShapeDtypeStruct((NP, HALF), jnp.float32),
          jax.ShapeDtypeStruct((NP,), jnp.float32),
      ],
  )(x, deg2)


def _matmul_body(y_ref, dinv_ref, w_ref, b_ref, a_ref, out_ref):
  di = dinv_ref[...][:, None]
  y0 = y_ref[0] * di
  y1 = y_ref[1] * di
  h = jnp.dot(y0, w_ref[:HALF, :], preferred_element_type=jnp.float32)
  h = h + jnp.dot(y1, w_ref[HALF:, :], preferred_element_type=jnp.float32)
  h = h + b_ref[...][None, :]
  out_ref[...] = jnp.where(h > 0, h, a_ref[...][None, :] * h)


def _matmul(y, dinv, W, b, prelu_a):
  return pl.pallas_call(
      _matmul_body,
      grid=(NP // _BLK,),
      in_specs=[
          pl.BlockSpec((2, _BLK, HALF), lambda i: (0, i, 0)),
          pl.BlockSpec((_BLK,), lambda i: (i,)),
          pl.BlockSpec((D_IN, D_H), lambda i: (0, 0)),
          pl.BlockSpec((D_H,), lambda i: (0,)),
          pl.BlockSpec((D_H,), lambda i: (0,)),
      ],
      out_specs=pl.BlockSpec((_BLK, D_H), lambda i: (i, 0)),
      out_shape=jax.ShapeDtypeStruct((N, D_H), jnp.float32),
  )(y, dinv, W, b, prelu_a)


@jax.jit
def kernel(x, edge_index, W, b, prelu_a):
  src = edge_index[0].astype(jnp.int32)
  dst = edge_index[1].astype(jnp.int32)
  npad = EP - E
  ar = jnp.arange(npad, dtype=jnp.int32)
  psrc = (ar * 911) % N
  pdst = N + (ar % (NACC - N))
  src_pad = jnp.concatenate([src, psrc])
  dst_pad = jnp.concatenate([dst, pdst])
  e5 = jnp.stack([src_pad, dst_pad]).reshape(2, NS, NPASS, WPASS, WIN)
  eagg = jnp.transpose(e5, (1, 2, 0, 3, 4)).reshape(NS * NPASS, 2, WPASS, WIN)
  edeg = dst_pad.reshape(NC * NS, WDEG, WIN)

  deg = _deg(edeg).reshape(2, NP)
  xs0, xs1, dinv = _scale(x, deg)
  y = _agg(eagg, xs0, xs1).reshape(2, NACC, HALF)
  return _matmul(y, dinv, W, b, prelu_a)
